# baseline (speedup 1.0000x reference)
import dataclasses
import functools

import numpy as np
import jax
import jax.numpy as jnp
from jax import lax
from jax.experimental import pallas as pl
from jax.experimental.pallas import tpu as pltpu
from jax.experimental.pallas import tpu_sc as plsc

BB, CC, ZZ, HH, WW = 2, 128, 8, 48, 48
HEADS, POINTS = 8, 4
GH, GW = ZZ * HH, WW
NQ = GH * GW
NT = BB * NQ
HD = CC // HEADS
TAPS = POINTS * 4
NROWS = NT * HEADS

TB = 512
NBLK = NQ // TB

NQH = NQ * HEADS
NWORK = 32
RW = NQH // NWORK
CR = 128
TPB = CR // HEADS
TPC = CR * TAPS
NGATH = TPC // 128
NCH = RW // CR
PKW = TPB * 2 * CC


def _lane_consts():
    px = np.zeros((2 * HEADS * POINTS, HEADS * TAPS), np.float32)
    py = np.zeros_like(px)
    rexp = np.zeros((HEADS * POINTS, HEADS * TAPS), np.float32)
    for i in range(HEADS * POINTS):
        for c in range(4):
            l = i * 4 + c
            px[2 * i, l] = 1.0
            py[2 * i + 1, l] = 1.0
            rexp[i, l] = 1.0
    gsum = np.zeros((HEADS * POINTS, HEADS * POINTS), np.float32)
    for i in range(HEADS * POINTS):
        for j in range(HEADS * POINTS):
            if i // POINTS == j // POINTS:
                gsum[i, j] = 1.0
    lanes = np.arange(HEADS * TAPS)
    corner = lanes % 4
    oxl = (corner % 2).astype(np.float32)[None, :]
    oyl = (corner // 2).astype(np.float32)[None, :]
    hl = (lanes // TAPS).astype(np.float32)[None, :]
    return px, py, rexp, gsum, oxl, oyl, hl


_PX, _PY, _REXP, _GSUM, _OXL, _OYL, _HL = _lane_consts()


def _prep_body(xt, wv, bv, wo, bo, wa, ba, pxm, pym, rexp, gsum, oxl, oyl, hl,
               val_o, pk_o):
    i = pl.program_id(0)
    qbase = i * TB
    xb = xt[0]
    dot = functools.partial(lax.dot_general,
                            precision=lax.Precision.HIGHEST,
                            preferred_element_type=jnp.float32)
    dnT = (((0,), (0,)), ((), ()))
    dnN = (((1,), (0,)), ((), ()))
    dot16 = functools.partial(lax.dot_general,
                              preferred_element_type=jnp.float32)
    xb16 = xb.astype(jnp.bfloat16)
    val_o[...] = dot16(xb16, wv[...].astype(jnp.bfloat16),
                       dimension_numbers=dnT) + bv[...]
    off = dot16(xb16, wo[...].astype(jnp.bfloat16),
                dimension_numbers=dnT) + bo[...]
    logits = dot16(xb16, wa[...].astype(jnp.bfloat16),
                   dimension_numbers=dnT) + ba[...]
    e = jnp.exp(logits)
    s = dot(e, gsum[...], dimension_numbers=dnN)
    attn = e / s
    attn128 = dot(attn, rexp[...], dimension_numbers=dnN)
    offx = dot(off, pxm[...], dimension_numbers=dnN)
    offy = dot(off, pym[...], dimension_numbers=dnN)
    q = qbase + lax.broadcasted_iota(jnp.int32, (TB, 1), 0)
    iw = (q % GW).astype(jnp.float32)
    ihw = (q // GW).astype(jnp.float32)
    px = offx + iw
    py = offy + ihw
    x0 = jnp.floor(px)
    y0 = jnp.floor(py)
    dx = px - x0
    dy = py - y0
    ox = oxl[...]
    oy = oyl[...]
    xi = x0 + ox
    yi = y0 + oy
    valid = ((xi >= 0) & (xi <= GW - 1) & (yi >= 0) & (yi <= GH - 1))
    xic = jnp.clip(xi, 0, GW - 1)
    yic = jnp.clip(yi, 0, GH - 1)
    lin = yic * GW + xic
    gidx = lin * HEADS + hl[...]
    pk_o[:, :CC] = gidx.astype(jnp.int32)
    wx = ox * dx + (1.0 - ox) * (1.0 - dx)
    wy = oy * dy + (1.0 - oy) * (1.0 - dy)
    wt = wx * wy * valid.astype(jnp.float32) * attn128
    pk_o[:, CC:] = lax.bitcast_convert_type(wt, jnp.int32)


def _full_spec(shape):
    nd = len(shape)
    return pl.BlockSpec(shape, lambda i: (0,) * nd)


def _prep_call(xt, wv, bv, wo, bo, wa, ba):
    consts = [jnp.asarray(a) for a in
              (_PX, _PY, _REXP, _GSUM, _OXL, _OYL, _HL)]
    return pl.pallas_call(
        _prep_body,
        grid=(NBLK,),
        in_specs=[
            pl.BlockSpec((1, CC, TB), lambda i: (0, 0, i)),
            _full_spec(wv.shape), _full_spec(bv.shape),
            _full_spec(wo.shape), _full_spec(bo.shape),
            _full_spec(wa.shape), _full_spec(ba.shape),
        ] + [_full_spec(c.shape) for c in consts],
        out_specs=[
            pl.BlockSpec((TB, CC), lambda i: (i, 0)),
            pl.BlockSpec((TB, 2 * CC), lambda i: (i, 0)),
        ],
        out_shape=[
            jax.ShapeDtypeStruct((NQ, CC), jnp.float32),
            jax.ShapeDtypeStruct((NQ, 2 * CC), jnp.int32),
        ],
    )(xt, wv, bv, wo, bo, wa, ba, *consts)


def _bcast(vec, j):
    return vec.at[jnp.full((HD,), j, jnp.int32)].get(mode="promise_in_bounds")


def _sc_body(val_hbm, pk_hbm, out_hbm,
             ld0, ld1, buf0, buf1, out0, out1,
             ls0, ls1, gs0, gs1, os0, os1):
    wid = lax.axis_index("s") * 2 + lax.axis_index("c")
    row_base = wid * RW
    tok_base = row_base // HEADS
    LD, BUF, OUT = (ld0, ld1), (buf0, buf1), (out0, out1)
    LS, GS, OS = (ls0, ls1), (gs0, gs1), (os0, os1)

    def issue_load(i, p):
        off = (tok_base + i * TPB) * (2 * CC)
        pltpu.async_copy(pk_hbm.at[pl.ds(off, PKW)], LD[p], LS[p])

    def wait_load(p):
        pltpu.make_async_copy(pk_hbm.at[pl.ds(0, PKW)], LD[p], LS[p]).wait()

    def issue_gathers(p):
        for g in range(NGATH):
            pltpu.async_copy(val_hbm.at[LD[p].at[pl.ds(g * 2 * CC, 128)]],
                             BUF[p].at[pl.ds(g * 128, 128)], GS[p])

    def wait_gathers(p):
        pltpu.make_async_copy(val_hbm.at[pl.ds(0, TPC)], BUF[p], GS[p]).wait()

    def wait_store(p):
        pltpu.make_async_copy(OUT[p], out_hbm.at[pl.ds(0, CR)], OS[p]).wait()

    def compute_store(i, p):
        @pl.loop(0, TPB)
        def _tok(m):
            ldb = m * (2 * CC) + CC
            bufb = m * CC
            for h in range(HEADS):
                w16 = plsc.bitcast(LD[p][pl.ds(ldb + h * HD, HD)],
                                   jnp.float32)
                a0 = _bcast(w16, 0) * BUF[p][bufb + h * HD + 0]
                a1 = _bcast(w16, 1) * BUF[p][bufb + h * HD + 1]
                a2 = _bcast(w16, 2) * BUF[p][bufb + h * HD + 2]
                a3 = _bcast(w16, 3) * BUF[p][bufb + h * HD + 3]
                for j in range(4, TAPS, 4):
                    a0 = a0 + _bcast(w16, j) * BUF[p][bufb + h * HD + j]
                    a1 = a1 + _bcast(w16, j + 1) * BUF[p][bufb + h * HD + j + 1]
                    a2 = a2 + _bcast(w16, j + 2) * BUF[p][bufb + h * HD + j + 2]
                    a3 = a3 + _bcast(w16, j + 3) * BUF[p][bufb + h * HD + j + 3]
                OUT[p][m * HEADS + h] = (a0 + a1) + (a2 + a3)

        pltpu.async_copy(OUT[p], out_hbm.at[pl.ds(row_base + i * CR, CR)],
                         OS[p])

    issue_load(0, 0)
    issue_load(1, 1)
    wait_load(0)
    issue_gathers(0)

    @pl.loop(0, NCH // 2)
    def _pair(k):
        for p in (0, 1):
            i = k * 2 + p
            q = 1 - p

            @pl.when(i + 1 < NCH)
            def _():
                wait_load(q)
                issue_gathers(q)

            wait_gathers(p)

            @pl.when(i >= 2)
            def _():
                wait_store(p)

            compute_store(i, p)

            @pl.when(i + 2 < NCH)
            def _():
                issue_load(i + 2, p)

    wait_store(0)
    wait_store(1)


@functools.cache
def _sc_gather_fn():
    mesh = plsc.VectorSubcoreMesh(core_axis_name="c", subcore_axis_name="s",
                                  num_cores=2, num_subcores=16)
    cp = pltpu.CompilerParams()
    if "needs_layout_passes" in pltpu.CompilerParams.__dataclass_fields__:
        cp = dataclasses.replace(cp, needs_layout_passes=False)
    if "use_tc_tiling_on_sc" in pltpu.CompilerParams.__dataclass_fields__:
        cp = dataclasses.replace(cp, use_tc_tiling_on_sc=False)
    return pl.kernel(
        _sc_body,
        out_type=jax.ShapeDtypeStruct((NQH, HD), jnp.float32),
        mesh=mesh,
        scratch_types=[
            pltpu.VMEM((PKW,), jnp.int32),
            pltpu.VMEM((PKW,), jnp.int32),
            pltpu.VMEM((TPC, HD), jnp.float32),
            pltpu.VMEM((TPC, HD), jnp.float32),
            pltpu.VMEM((CR, HD), jnp.float32),
            pltpu.VMEM((CR, HD), jnp.float32),
            pltpu.SemaphoreType.DMA,
            pltpu.SemaphoreType.DMA,
            pltpu.SemaphoreType.DMA,
            pltpu.SemaphoreType.DMA,
            pltpu.SemaphoreType.DMA,
            pltpu.SemaphoreType.DMA,
        ],
        compiler_params=cp,
    )


def _out_body(rows, wout, bout, o_ref):
    ot = lax.dot_general(wout[...].astype(jnp.bfloat16),
                         rows[...].astype(jnp.bfloat16),
                         dimension_numbers=(((0,), (1,)), ((), ())),
                         preferred_element_type=jnp.float32)
    o_ref[0] = ot + bout[...]


def _out_call(rows, wout, bout):
    return pl.pallas_call(
        _out_body,
        grid=(NBLK,),
        in_specs=[
            pl.BlockSpec((TB, CC), lambda i: (i, 0)),
            _full_spec(wout.shape), _full_spec(bout.shape),
        ],
        out_specs=pl.BlockSpec((1, CC, TB), lambda i: (0, 0, i)),
        out_shape=jax.ShapeDtypeStruct((1, CC, NQ), jnp.float32),
    )(rows, wout, bout)


@jax.jit
def kernel(x, W_off, b_off, W_attn, b_attn, W_val, b_val, W_out, b_out):
    xt = x.reshape(BB, CC, NQ)
    sc = _sc_gather_fn()
    prepped = [
        _prep_call(xt[b:b + 1], W_val, b_val.reshape(1, CC),
                   W_off, b_off.reshape(1, -1),
                   W_attn, b_attn.reshape(1, -1))
        for b in range(BB)
    ]
    sampled = [sc(v.reshape(NQH, HD), p.reshape(-1)) for v, p in prepped]
    outs = [_out_call(s.reshape(NQ, CC), W_out, b_out.reshape(CC, 1))
            for s in sampled]
    return jnp.concatenate(outs, axis=0).reshape(BB, CC, ZZ, HH, WW)


# speedup vs baseline: 1.0011x; 1.0011x over previous
import dataclasses
import functools

import numpy as np
import jax
import jax.numpy as jnp
from jax import lax
from jax.experimental import pallas as pl
from jax.experimental.pallas import tpu as pltpu
from jax.experimental.pallas import tpu_sc as plsc

BB, CC, ZZ, HH, WW = 2, 128, 8, 48, 48
HEADS, POINTS = 8, 4
GH, GW = ZZ * HH, WW
NQ = GH * GW
NT = BB * NQ
HD = CC // HEADS
TAPS = POINTS * 4
NROWS = NT * HEADS

TB = 512
NBLK = NQ // TB

NQH = NQ * HEADS
NWORK = 32
RW = NQH // NWORK
CR = 128
TPB = CR // HEADS
TPC = CR * TAPS
NGATH = TPC // 128
NCH = RW // CR
PKW = TPB * 2 * CC


def _lane_consts():
    px = np.zeros((2 * HEADS * POINTS, HEADS * TAPS), np.float32)
    py = np.zeros_like(px)
    rexp = np.zeros((HEADS * POINTS, HEADS * TAPS), np.float32)
    for i in range(HEADS * POINTS):
        for c in range(4):
            l = i * 4 + c
            px[2 * i, l] = 1.0
            py[2 * i + 1, l] = 1.0
            rexp[i, l] = 1.0
    gsum = np.zeros((HEADS * POINTS, HEADS * POINTS), np.float32)
    for i in range(HEADS * POINTS):
        for j in range(HEADS * POINTS):
            if i // POINTS == j // POINTS:
                gsum[i, j] = 1.0
    lanes = np.arange(HEADS * TAPS)
    corner = lanes % 4
    oxl = (corner % 2).astype(np.float32)[None, :]
    oyl = (corner // 2).astype(np.float32)[None, :]
    hl = (lanes // TAPS).astype(np.float32)[None, :]
    return px, py, rexp, gsum, oxl, oyl, hl


_PX, _PY, _REXP, _GSUM, _OXL, _OYL, _HL = _lane_consts()


def _prep_body(xt, wv, bv, wo, bo, wa, ba, pxm, pym, rexp, gsum, oxl, oyl, hl,
               val_o, pk_o):
    i = pl.program_id(0)
    qbase = i * TB
    xb = xt[0]
    dot = functools.partial(lax.dot_general,
                            precision=lax.Precision.HIGHEST,
                            preferred_element_type=jnp.float32)
    dnT = (((0,), (0,)), ((), ()))
    dnN = (((1,), (0,)), ((), ()))
    dot16 = functools.partial(lax.dot_general,
                              preferred_element_type=jnp.float32)
    xb16 = xb.astype(jnp.bfloat16)
    val_o[...] = dot16(xb16, wv[...].astype(jnp.bfloat16),
                       dimension_numbers=dnT) + bv[...]
    off = dot16(xb16, wo[...].astype(jnp.bfloat16),
                dimension_numbers=dnT) + bo[...]
    logits = dot16(xb16, wa[...].astype(jnp.bfloat16),
                   dimension_numbers=dnT) + ba[...]
    e = jnp.exp(logits)
    s = dot(e, gsum[...], dimension_numbers=dnN)
    attn = e / s
    attn128 = dot(attn, rexp[...], dimension_numbers=dnN)
    offx = dot(off, pxm[...], dimension_numbers=dnN)
    offy = dot(off, pym[...], dimension_numbers=dnN)
    q = qbase + lax.broadcasted_iota(jnp.int32, (TB, 1), 0)
    iw = (q % GW).astype(jnp.float32)
    ihw = (q // GW).astype(jnp.float32)
    px = offx + iw
    py = offy + ihw
    x0 = jnp.floor(px)
    y0 = jnp.floor(py)
    dx = px - x0
    dy = py - y0
    ox = oxl[...]
    oy = oyl[...]
    xi = x0 + ox
    yi = y0 + oy
    valid = ((xi >= 0) & (xi <= GW - 1) & (yi >= 0) & (yi <= GH - 1))
    xic = jnp.clip(xi, 0, GW - 1)
    yic = jnp.clip(yi, 0, GH - 1)
    lin = yic * GW + xic
    gidx = lin * HEADS + hl[...]
    pk_o[:, :CC] = gidx.astype(jnp.int32)
    wx = ox * dx + (1.0 - ox) * (1.0 - dx)
    wy = oy * dy + (1.0 - oy) * (1.0 - dy)
    wt = wx * wy * valid.astype(jnp.float32) * attn128
    pk_o[:, CC:] = lax.bitcast_convert_type(wt, jnp.int32)


def _full_spec(shape):
    nd = len(shape)
    return pl.BlockSpec(shape, lambda i: (0,) * nd)


def _prep_call(xt, wv, bv, wo, bo, wa, ba):
    consts = [jnp.asarray(a) for a in
              (_PX, _PY, _REXP, _GSUM, _OXL, _OYL, _HL)]
    return pl.pallas_call(
        _prep_body,
        grid=(NBLK,),
        in_specs=[
            pl.BlockSpec((1, CC, TB), lambda i: (0, 0, i)),
            _full_spec(wv.shape), _full_spec(bv.shape),
            _full_spec(wo.shape), _full_spec(bo.shape),
            _full_spec(wa.shape), _full_spec(ba.shape),
        ] + [_full_spec(c.shape) for c in consts],
        out_specs=[
            pl.BlockSpec((TB, CC), lambda i: (i, 0)),
            pl.BlockSpec((TB, 2 * CC), lambda i: (i, 0)),
        ],
        out_shape=[
            jax.ShapeDtypeStruct((NQ, CC), jnp.float32),
            jax.ShapeDtypeStruct((NQ, 2 * CC), jnp.int32),
        ],
    )(xt, wv, bv, wo, bo, wa, ba, *consts)


def _bcast(vec, j):
    return vec.at[jnp.full((HD,), j, jnp.int32)].get(mode="promise_in_bounds")


def _sc_body(val_hbm, pk_hbm, out_hbm,
             ld0, ld1, buf0, buf1, out0, out1,
             ls0, ls1, gs0, gs1, os0, os1):
    wid = lax.axis_index("s") * 2 + lax.axis_index("c")
    row_base = wid * RW
    tok_base = row_base // HEADS
    LD, BUF, OUT = (ld0, ld1), (buf0, buf1), (out0, out1)
    LS, GS, OS = (ls0, ls1), (gs0, gs1), (os0, os1)

    def issue_load(i, p):
        off = (tok_base + i * TPB) * (2 * CC)
        pltpu.async_copy(pk_hbm.at[pl.ds(off, PKW)], LD[p], LS[p])

    def wait_load(p):
        pltpu.make_async_copy(pk_hbm.at[pl.ds(0, PKW)], LD[p], LS[p]).wait()

    def issue_gathers(p):
        for g in range(NGATH):
            pltpu.async_copy(val_hbm.at[LD[p].at[pl.ds(g * 2 * CC, 128)]],
                             BUF[p].at[pl.ds(g * 128, 128)], GS[p])

    def wait_gathers(p):
        pltpu.make_async_copy(val_hbm.at[pl.ds(0, TPC)], BUF[p], GS[p]).wait()

    def wait_store(p):
        pltpu.make_async_copy(OUT[p], out_hbm.at[pl.ds(0, CR)], OS[p]).wait()

    def compute_store(i, p):
        @pl.loop(0, TPB)
        def _tok(m):
            ldb = m * (2 * CC) + CC
            bufb = m * CC
            for h in range(HEADS):
                w16 = plsc.bitcast(LD[p][pl.ds(ldb + h * HD, HD)],
                                   jnp.float32)
                a0 = _bcast(w16, 0) * BUF[p][bufb + h * HD + 0]
                a1 = _bcast(w16, 1) * BUF[p][bufb + h * HD + 1]
                a2 = _bcast(w16, 2) * BUF[p][bufb + h * HD + 2]
                a3 = _bcast(w16, 3) * BUF[p][bufb + h * HD + 3]
                for j in range(4, TAPS, 4):
                    a0 = a0 + _bcast(w16, j) * BUF[p][bufb + h * HD + j]
                    a1 = a1 + _bcast(w16, j + 1) * BUF[p][bufb + h * HD + j + 1]
                    a2 = a2 + _bcast(w16, j + 2) * BUF[p][bufb + h * HD + j + 2]
                    a3 = a3 + _bcast(w16, j + 3) * BUF[p][bufb + h * HD + j + 3]
                OUT[p][m * HEADS + h] = (a0 + a1) + (a2 + a3)

        pltpu.async_copy(OUT[p], out_hbm.at[pl.ds(row_base + i * CR, CR)],
                         OS[p])

    issue_load(0, 0)
    issue_load(1, 1)
    wait_load(0)
    issue_gathers(0)

    @pl.loop(0, NCH // 2)
    def _pair(k):
        for p in (0, 1):
            i = k * 2 + p
            q = 1 - p

            @pl.when(i + 1 < NCH)
            def _():
                wait_load(q)
                issue_gathers(q)

            wait_gathers(p)

            @pl.when(i >= 2)
            def _():
                wait_store(p)

            compute_store(i, p)

            @pl.when(i + 2 < NCH)
            def _():
                issue_load(i + 2, p)

    wait_store(0)
    wait_store(1)


@functools.cache
def _sc_gather_fn():
    mesh = plsc.VectorSubcoreMesh(core_axis_name="c", subcore_axis_name="s",
                                  num_cores=2, num_subcores=16)
    cp = pltpu.CompilerParams()
    if "needs_layout_passes" in pltpu.CompilerParams.__dataclass_fields__:
        cp = dataclasses.replace(cp, needs_layout_passes=False)
    if "use_tc_tiling_on_sc" in pltpu.CompilerParams.__dataclass_fields__:
        cp = dataclasses.replace(cp, use_tc_tiling_on_sc=False)
    return pl.kernel(
        _sc_body,
        out_type=jax.ShapeDtypeStruct((NQH, HD), jnp.float32),
        mesh=mesh,
        scratch_types=[
            pltpu.VMEM((PKW,), jnp.int32),
            pltpu.VMEM((PKW,), jnp.int32),
            pltpu.VMEM((TPC, HD), jnp.float32),
            pltpu.VMEM((TPC, HD), jnp.float32),
            pltpu.VMEM((CR, HD), jnp.float32),
            pltpu.VMEM((CR, HD), jnp.float32),
            pltpu.SemaphoreType.DMA,
            pltpu.SemaphoreType.DMA,
            pltpu.SemaphoreType.DMA,
            pltpu.SemaphoreType.DMA,
            pltpu.SemaphoreType.DMA,
            pltpu.SemaphoreType.DMA,
        ],
        compiler_params=cp,
    )


def _out_body(rows, wout, bout, o_ref):
    ot = lax.dot_general(wout[...].astype(jnp.bfloat16),
                         rows[...].astype(jnp.bfloat16),
                         dimension_numbers=(((0,), (1,)), ((), ())),
                         preferred_element_type=jnp.float32)
    o_ref[0] = ot + bout[...]


def _out_call(rows, wout, bout):
    return pl.pallas_call(
        _out_body,
        grid=(NBLK,),
        in_specs=[
            pl.BlockSpec((TB, CC), lambda i: (i, 0)),
            _full_spec(wout.shape), _full_spec(bout.shape),
        ],
        out_specs=pl.BlockSpec((1, CC, TB), lambda i: (0, 0, i)),
        out_shape=jax.ShapeDtypeStruct((1, CC, NQ), jnp.float32),
    )(rows, wout, bout)


@jax.jit
def kernel(x, W_off, b_off, W_attn, b_attn, W_val, b_val, W_out, b_out):
    xt = x.reshape(BB, CC, NQ)
    sc = _sc_gather_fn()
    sampled = []
    for b in range(BB):
        v, p = _prep_call(xt[b:b + 1], W_val, b_val.reshape(1, CC),
                          W_off, b_off.reshape(1, -1),
                          W_attn, b_attn.reshape(1, -1))
        sampled.append(sc(v.reshape(NQH, HD), p.reshape(-1)))
    outs = [_out_call(s.reshape(NQ, CC), W_out, b_out.reshape(CC, 1))
            for s in sampled]
    return jnp.concatenate(outs, axis=0).reshape(BB, CC, ZZ, HH, WW)


# speedup vs baseline: 1.0941x; 1.0930x over previous
import dataclasses
import functools

import numpy as np
import jax
import jax.numpy as jnp
from jax import lax
from jax.experimental import pallas as pl
from jax.experimental.pallas import tpu as pltpu
from jax.experimental.pallas import tpu_sc as plsc

BB, CC, ZZ, HH, WW = 2, 128, 8, 48, 48
HEADS, POINTS = 8, 4
GH, GW = ZZ * HH, WW
NQ = GH * GW
NT = BB * NQ
HD = CC // HEADS
TAPS = POINTS * 4
NROWS = NT * HEADS

TB = 512
NBLK = NQ // TB

NQH = NQ * HEADS
NWORK = 32
RW = NROWS // NWORK
TKW = NT // NWORK
CR = 128
TPB = CR // HEADS
TPC = CR * TAPS
NGATH = TPC // 128
NCH = RW // CR
HPKW = TPB * CC


def _lane_consts():
    px = np.zeros((2 * HEADS * POINTS, HEADS * TAPS), np.float32)
    py = np.zeros_like(px)
    rexp = np.zeros((HEADS * POINTS, HEADS * TAPS), np.float32)
    for i in range(HEADS * POINTS):
        for c in range(4):
            l = i * 4 + c
            px[2 * i, l] = 1.0
            py[2 * i + 1, l] = 1.0
            rexp[i, l] = 1.0
    gsum = np.zeros((HEADS * POINTS, HEADS * POINTS), np.float32)
    for i in range(HEADS * POINTS):
        for j in range(HEADS * POINTS):
            if i // POINTS == j // POINTS:
                gsum[i, j] = 1.0
    lanes = np.arange(HEADS * TAPS)
    corner = lanes % 4
    oxl = (corner % 2).astype(np.float32)[None, :]
    oyl = (corner // 2).astype(np.float32)[None, :]
    hl = (lanes // TAPS).astype(np.float32)[None, :]
    return px, py, rexp, gsum, oxl, oyl, hl


_PX, _PY, _REXP, _GSUM, _OXL, _OYL, _HL = _lane_consts()


def _prep_body(xt, wv, bv, wo, bo, wa, ba, pxm, pym, rexp, gsum, oxl, oyl, hl,
               val_o, pk_o):
    i = pl.program_id(0)
    b = i // NBLK
    qbase = (i % NBLK) * TB
    xb = xt[0]
    dot = functools.partial(lax.dot_general,
                            precision=lax.Precision.HIGHEST,
                            preferred_element_type=jnp.float32)
    dnT = (((0,), (0,)), ((), ()))
    dnN = (((1,), (0,)), ((), ()))
    dot16 = functools.partial(lax.dot_general,
                              preferred_element_type=jnp.float32)
    xb16 = xb.astype(jnp.bfloat16)
    val_o[...] = dot16(xb16, wv[...].astype(jnp.bfloat16),
                       dimension_numbers=dnT) + bv[...]
    off = dot16(xb16, wo[...].astype(jnp.bfloat16),
                dimension_numbers=dnT) + bo[...]
    logits = dot16(xb16, wa[...].astype(jnp.bfloat16),
                   dimension_numbers=dnT) + ba[...]
    e = jnp.exp(logits)
    s = dot(e, gsum[...], dimension_numbers=dnN)
    attn = e / s
    attn128 = dot(attn, rexp[...], dimension_numbers=dnN)
    offx = dot(off, pxm[...], dimension_numbers=dnN)
    offy = dot(off, pym[...], dimension_numbers=dnN)
    q = qbase + lax.broadcasted_iota(jnp.int32, (TB, 1), 0)
    iw = (q % GW).astype(jnp.float32)
    ihw = (q // GW).astype(jnp.float32)
    px = offx + iw
    py = offy + ihw
    x0 = jnp.floor(px)
    y0 = jnp.floor(py)
    dx = px - x0
    dy = py - y0
    ox = oxl[...]
    oy = oyl[...]
    xi = x0 + ox
    yi = y0 + oy
    valid = ((xi >= 0) & (xi <= GW - 1) & (yi >= 0) & (yi <= GH - 1))
    xic = jnp.clip(xi, 0, GW - 1)
    yic = jnp.clip(yi, 0, GH - 1)
    lin = yic * GW + xic
    gidx = lin * HEADS + hl[...] + (b * NQH).astype(jnp.float32)
    pk_o[:TB] = gidx.astype(jnp.int32)
    wx = ox * dx + (1.0 - ox) * (1.0 - dx)
    wy = oy * dy + (1.0 - oy) * (1.0 - dy)
    wt = wx * wy * valid.astype(jnp.float32) * attn128
    pk_o[TB:] = lax.bitcast_convert_type(wt, jnp.int32)


def _full_spec(shape):
    nd = len(shape)
    return pl.BlockSpec(shape, lambda i: (0,) * nd)


def _prep_call(xt, wv, bv, wo, bo, wa, ba):
    consts = [jnp.asarray(a) for a in
              (_PX, _PY, _REXP, _GSUM, _OXL, _OYL, _HL)]
    return pl.pallas_call(
        _prep_body,
        grid=(BB * NBLK,),
        in_specs=[
            pl.BlockSpec((1, CC, TB), lambda i: (i // NBLK, 0, i % NBLK)),
            _full_spec(wv.shape), _full_spec(bv.shape),
            _full_spec(wo.shape), _full_spec(bo.shape),
            _full_spec(wa.shape), _full_spec(ba.shape),
        ] + [_full_spec(c.shape) for c in consts],
        out_specs=[
            pl.BlockSpec((TB, CC), lambda i: (i, 0)),
            pl.BlockSpec((2 * TB, CC), lambda i: (i, 0)),
        ],
        out_shape=[
            jax.ShapeDtypeStruct((NT, CC), jnp.float32),
            jax.ShapeDtypeStruct((2 * NT, CC), jnp.int32),
        ],
    )(xt, wv, bv, wo, bo, wa, ba, *consts)


def _bcast(vec, j):
    return vec.at[jnp.full((HD,), j, jnp.int32)].get(mode="promise_in_bounds")


def _sc_body(val_hbm, pk_hbm, out_hbm,
             ld0, ld1, buf0, buf1, out0, out1,
             ls0, ls1, gs0, gs1, os0, os1):
    wid = lax.axis_index("s") * 2 + lax.axis_index("c")
    row_base = wid * RW
    tok_base = wid * TKW
    LD, BUF, OUT = (ld0, ld1), (buf0, buf1), (out0, out1)
    LS, GS, OS = (ls0, ls1), (gs0, gs1), (os0, os1)

    def issue_load(i, p):
        t0 = tok_base + i * TPB
        g = t0 // TB
        r = t0 % TB
        idx_off = (g * 2 * TB + r) * CC
        wt_off = (g * 2 * TB + TB + r) * CC
        pltpu.async_copy(pk_hbm.at[pl.ds(idx_off, HPKW)],
                         LD[p].at[pl.ds(0, HPKW)], LS[p])
        pltpu.async_copy(pk_hbm.at[pl.ds(wt_off, HPKW)],
                         LD[p].at[pl.ds(HPKW, HPKW)], LS[p])

    def wait_load(p):
        pltpu.make_async_copy(pk_hbm.at[pl.ds(0, 2 * HPKW)], LD[p],
                              LS[p]).wait()

    def issue_gathers(p):
        for g in range(NGATH):
            pltpu.async_copy(val_hbm.at[LD[p].at[pl.ds(g * 128, 128)]],
                             BUF[p].at[pl.ds(g * 128, 128)], GS[p])

    def wait_gathers(p):
        pltpu.make_async_copy(val_hbm.at[pl.ds(0, TPC)], BUF[p], GS[p]).wait()

    def wait_store(p):
        pltpu.make_async_copy(OUT[p], out_hbm.at[pl.ds(0, CR)], OS[p]).wait()

    def compute_store(i, p):
        @pl.loop(0, TPB)
        def _tok(m):
            ldb = HPKW + m * CC
            bufb = m * CC
            for h in range(HEADS):
                w16 = plsc.bitcast(LD[p][pl.ds(ldb + h * HD, HD)],
                                   jnp.float32)
                a0 = _bcast(w16, 0) * BUF[p][bufb + h * HD + 0]
                a1 = _bcast(w16, 1) * BUF[p][bufb + h * HD + 1]
                a2 = _bcast(w16, 2) * BUF[p][bufb + h * HD + 2]
                a3 = _bcast(w16, 3) * BUF[p][bufb + h * HD + 3]
                for j in range(4, TAPS, 4):
                    a0 = a0 + _bcast(w16, j) * BUF[p][bufb + h * HD + j]
                    a1 = a1 + _bcast(w16, j + 1) * BUF[p][bufb + h * HD + j + 1]
                    a2 = a2 + _bcast(w16, j + 2) * BUF[p][bufb + h * HD + j + 2]
                    a3 = a3 + _bcast(w16, j + 3) * BUF[p][bufb + h * HD + j + 3]
                OUT[p][m * HEADS + h] = (a0 + a1) + (a2 + a3)

        pltpu.async_copy(OUT[p], out_hbm.at[pl.ds(row_base + i * CR, CR)],
                         OS[p])

    issue_load(0, 0)
    issue_load(1, 1)
    wait_load(0)
    issue_gathers(0)

    @pl.loop(0, NCH // 2)
    def _pair(k):
        for p in (0, 1):
            i = k * 2 + p
            q = 1 - p

            @pl.when(i + 1 < NCH)
            def _():
                wait_load(q)
                issue_gathers(q)

            wait_gathers(p)

            @pl.when(i >= 2)
            def _():
                wait_store(p)

            compute_store(i, p)

            @pl.when(i + 2 < NCH)
            def _():
                issue_load(i + 2, p)

    wait_store(0)
    wait_store(1)


@functools.cache
def _sc_gather_fn():
    mesh = plsc.VectorSubcoreMesh(core_axis_name="c", subcore_axis_name="s",
                                  num_cores=2, num_subcores=16)
    cp = pltpu.CompilerParams()
    if "needs_layout_passes" in pltpu.CompilerParams.__dataclass_fields__:
        cp = dataclasses.replace(cp, needs_layout_passes=False)
    if "use_tc_tiling_on_sc" in pltpu.CompilerParams.__dataclass_fields__:
        cp = dataclasses.replace(cp, use_tc_tiling_on_sc=False)
    return pl.kernel(
        _sc_body,
        out_type=jax.ShapeDtypeStruct((NROWS, HD), jnp.float32),
        mesh=mesh,
        scratch_types=[
            pltpu.VMEM((2 * HPKW,), jnp.int32),
            pltpu.VMEM((2 * HPKW,), jnp.int32),
            pltpu.VMEM((TPC, HD), jnp.float32),
            pltpu.VMEM((TPC, HD), jnp.float32),
            pltpu.VMEM((CR, HD), jnp.float32),
            pltpu.VMEM((CR, HD), jnp.float32),
            pltpu.SemaphoreType.DMA,
            pltpu.SemaphoreType.DMA,
            pltpu.SemaphoreType.DMA,
            pltpu.SemaphoreType.DMA,
            pltpu.SemaphoreType.DMA,
            pltpu.SemaphoreType.DMA,
        ],
        compiler_params=cp,
    )


def _out_body(rows, wout, bout, o_ref):
    ot = lax.dot_general(wout[...].astype(jnp.bfloat16),
                         rows[...].astype(jnp.bfloat16),
                         dimension_numbers=(((0,), (1,)), ((), ())),
                         preferred_element_type=jnp.float32)
    o_ref[0] = ot + bout[...]


def _out_call(rows, wout, bout):
    return pl.pallas_call(
        _out_body,
        grid=(BB * NBLK,),
        in_specs=[
            pl.BlockSpec((TB, CC), lambda i: (i, 0)),
            _full_spec(wout.shape), _full_spec(bout.shape),
        ],
        out_specs=pl.BlockSpec((1, CC, TB), lambda i: (i // NBLK, 0, i % NBLK)),
        out_shape=jax.ShapeDtypeStruct((BB, CC, NQ), jnp.float32),
    )(rows, wout, bout)


@jax.jit
def kernel(x, W_off, b_off, W_attn, b_attn, W_val, b_val, W_out, b_out):
    xt = x.reshape(BB, CC, NQ)
    value, packed = _prep_call(
        xt, W_val, b_val.reshape(1, CC),
        W_off, b_off.reshape(1, -1),
        W_attn, b_attn.reshape(1, -1))
    sampled = _sc_gather_fn()(value.reshape(NROWS, HD), packed.reshape(-1))
    out_t = _out_call(sampled.reshape(NT, CC), W_out, b_out.reshape(CC, 1))
    return out_t.reshape(BB, CC, ZZ, HH, WW)


# speedup vs baseline: 1.1998x; 1.0966x over previous
import dataclasses
import functools

import numpy as np
import jax
import jax.numpy as jnp
from jax import lax
from jax.experimental import pallas as pl
from jax.experimental.pallas import tpu as pltpu
from jax.experimental.pallas import tpu_sc as plsc

BB, CC, ZZ, HH, WW = 2, 128, 8, 48, 48
HEADS, POINTS = 8, 4
GH, GW = ZZ * HH, WW
NQ = GH * GW
NT = BB * NQ
HD = CC // HEADS
TAPS = POINTS * 4
NROWS = NT * HEADS

TB = 1024
NBLK = NQ // TB

NQH = NQ * HEADS
NWORK = 32
RW = NROWS // NWORK
TKW = NT // NWORK
CR = 128
TPB = CR // HEADS
TPC = CR * TAPS
NGATH = TPC // 128
NCH = RW // CR
HPKW = TPB * CC


def _lane_consts():
    px = np.zeros((2 * HEADS * POINTS, HEADS * TAPS), np.float32)
    py = np.zeros_like(px)
    rexp = np.zeros((HEADS * POINTS, HEADS * TAPS), np.float32)
    for i in range(HEADS * POINTS):
        for c in range(4):
            l = i * 4 + c
            px[2 * i, l] = 1.0
            py[2 * i + 1, l] = 1.0
            rexp[i, l] = 1.0
    gsum = np.zeros((HEADS * POINTS, HEADS * POINTS), np.float32)
    for i in range(HEADS * POINTS):
        for j in range(HEADS * POINTS):
            if i // POINTS == j // POINTS:
                gsum[i, j] = 1.0
    lanes = np.arange(HEADS * TAPS)
    corner = lanes % 4
    oxl = (corner % 2).astype(np.float32)[None, :]
    oyl = (corner // 2).astype(np.float32)[None, :]
    hl = (lanes // TAPS).astype(np.float32)[None, :]
    return px, py, rexp, gsum, oxl, oyl, hl


_PX, _PY, _REXP, _GSUM, _OXL, _OYL, _HL = _lane_consts()


def _prep_body(xt, wv, bv, wo, bo, wa, ba, pxm, pym, rexp, gsum, oxl, oyl, hl,
               val_o, pk_o):
    i = pl.program_id(0)
    b = i // NBLK
    qbase = (i % NBLK) * TB
    xb = xt[0]
    dot = functools.partial(lax.dot_general,
                            precision=lax.Precision.HIGHEST,
                            preferred_element_type=jnp.float32)
    dnT = (((0,), (0,)), ((), ()))
    dnN = (((1,), (0,)), ((), ()))
    dot16 = functools.partial(lax.dot_general,
                              preferred_element_type=jnp.float32)
    xb16 = xb.astype(jnp.bfloat16)
    val_o[...] = dot16(xb16, wv[...], dimension_numbers=dnT) + bv[...]
    off = dot16(xb16, wo[...], dimension_numbers=dnT) + bo[...]
    logits = dot16(xb16, wa[...], dimension_numbers=dnT) + ba[...]
    e = jnp.exp(logits)
    s = dot(e, gsum[...], dimension_numbers=dnN)
    attn = e / s
    attn128 = dot(attn, rexp[...], dimension_numbers=dnN)
    offx = dot(off, pxm[...], dimension_numbers=dnN)
    offy = dot(off, pym[...], dimension_numbers=dnN)
    q = qbase + lax.broadcasted_iota(jnp.int32, (TB, 1), 0)
    iw = (q % GW).astype(jnp.float32)
    ihw = (q // GW).astype(jnp.float32)
    px = offx + iw
    py = offy + ihw
    x0 = jnp.floor(px)
    y0 = jnp.floor(py)
    dx = px - x0
    dy = py - y0
    ox = oxl[...]
    oy = oyl[...]
    xi = x0 + ox
    yi = y0 + oy
    valid = ((xi >= 0) & (xi <= GW - 1) & (yi >= 0) & (yi <= GH - 1))
    xic = jnp.clip(xi, 0, GW - 1)
    yic = jnp.clip(yi, 0, GH - 1)
    lin = yic * GW + xic
    gidx = lin * HEADS + hl[...] + (b * NQH).astype(jnp.float32)
    pk_o[:TB] = gidx.astype(jnp.int32)
    wx = ox * dx + (1.0 - ox) * (1.0 - dx)
    wy = oy * dy + (1.0 - oy) * (1.0 - dy)
    wt = wx * wy * valid.astype(jnp.float32) * attn128
    pk_o[TB:] = lax.bitcast_convert_type(wt, jnp.int32)


def _full_spec(shape):
    nd = len(shape)
    return pl.BlockSpec(shape, lambda i: (0,) * nd)


def _prep_call(xt, wv, bv, wo, bo, wa, ba):
    consts = [jnp.asarray(a) for a in
              (_PX, _PY, _REXP, _GSUM, _OXL, _OYL, _HL)]
    return pl.pallas_call(
        _prep_body,
        grid=(BB * NBLK,),
        in_specs=[
            pl.BlockSpec((1, CC, TB), lambda i: (i // NBLK, 0, i % NBLK)),
            _full_spec(wv.shape), _full_spec(bv.shape),
            _full_spec(wo.shape), _full_spec(bo.shape),
            _full_spec(wa.shape), _full_spec(ba.shape),
        ] + [_full_spec(c.shape) for c in consts],
        out_specs=[
            pl.BlockSpec((TB, CC), lambda i: (i, 0)),
            pl.BlockSpec((2 * TB, CC), lambda i: (i, 0)),
        ],
        out_shape=[
            jax.ShapeDtypeStruct((NT, CC), jnp.float32),
            jax.ShapeDtypeStruct((2 * NT, CC), jnp.int32),
        ],
    )(xt, wv, bv, wo, bo, wa, ba, *consts)


def _bcast(vec, j):
    return vec.at[jnp.full((HD,), j, jnp.int32)].get(mode="promise_in_bounds")


def _sc_body(val_hbm, pk_hbm, out_hbm,
             ld0, ld1, buf0, buf1, out0, out1,
             ls0, ls1, gs0, gs1, os0, os1):
    wid = lax.axis_index("s") * 2 + lax.axis_index("c")
    row_base = wid * RW
    tok_base = wid * TKW
    LD, BUF, OUT = (ld0, ld1), (buf0, buf1), (out0, out1)
    LS, GS, OS = (ls0, ls1), (gs0, gs1), (os0, os1)

    def issue_load(i, p):
        t0 = tok_base + i * TPB
        g = t0 // TB
        r = t0 % TB
        idx_off = (g * 2 * TB + r) * CC
        wt_off = (g * 2 * TB + TB + r) * CC
        pltpu.async_copy(pk_hbm.at[pl.ds(idx_off, HPKW)],
                         LD[p].at[pl.ds(0, HPKW)], LS[p])
        pltpu.async_copy(pk_hbm.at[pl.ds(wt_off, HPKW)],
                         LD[p].at[pl.ds(HPKW, HPKW)], LS[p])

    def wait_load(p):
        pltpu.make_async_copy(pk_hbm.at[pl.ds(0, 2 * HPKW)], LD[p],
                              LS[p]).wait()

    def issue_gathers(p):
        for g in range(NGATH):
            pltpu.async_copy(val_hbm.at[LD[p].at[pl.ds(g * 128, 128)]],
                             BUF[p].at[pl.ds(g * 128, 128)], GS[p])

    def wait_gathers(p):
        pltpu.make_async_copy(val_hbm.at[pl.ds(0, TPC)], BUF[p], GS[p]).wait()

    def wait_store(p):
        pltpu.make_async_copy(OUT[p], out_hbm.at[pl.ds(0, CR)], OS[p]).wait()

    def compute_store(i, p):
        @pl.loop(0, TPB)
        def _tok(m):
            ldb = HPKW + m * CC
            bufb = m * CC
            for h in range(HEADS):
                w16 = plsc.bitcast(LD[p][pl.ds(ldb + h * HD, HD)],
                                   jnp.float32)
                a0 = _bcast(w16, 0) * BUF[p][bufb + h * HD + 0]
                a1 = _bcast(w16, 1) * BUF[p][bufb + h * HD + 1]
                a2 = _bcast(w16, 2) * BUF[p][bufb + h * HD + 2]
                a3 = _bcast(w16, 3) * BUF[p][bufb + h * HD + 3]
                for j in range(4, TAPS, 4):
                    a0 = a0 + _bcast(w16, j) * BUF[p][bufb + h * HD + j]
                    a1 = a1 + _bcast(w16, j + 1) * BUF[p][bufb + h * HD + j + 1]
                    a2 = a2 + _bcast(w16, j + 2) * BUF[p][bufb + h * HD + j + 2]
                    a3 = a3 + _bcast(w16, j + 3) * BUF[p][bufb + h * HD + j + 3]
                OUT[p][m * HEADS + h] = (a0 + a1) + (a2 + a3)

        pltpu.async_copy(OUT[p], out_hbm.at[pl.ds(row_base + i * CR, CR)],
                         OS[p])

    issue_load(0, 0)
    issue_load(1, 1)
    wait_load(0)
    issue_gathers(0)

    @pl.loop(0, NCH // 2)
    def _pair(k):
        for p in (0, 1):
            i = k * 2 + p
            q = 1 - p

            @pl.when(i + 1 < NCH)
            def _():
                wait_load(q)
                issue_gathers(q)

            wait_gathers(p)

            @pl.when(i >= 2)
            def _():
                wait_store(p)

            compute_store(i, p)

            @pl.when(i + 2 < NCH)
            def _():
                issue_load(i + 2, p)

    wait_store(0)
    wait_store(1)


@functools.cache
def _sc_gather_fn():
    mesh = plsc.VectorSubcoreMesh(core_axis_name="c", subcore_axis_name="s",
                                  num_cores=2, num_subcores=16)
    cp = pltpu.CompilerParams()
    if "needs_layout_passes" in pltpu.CompilerParams.__dataclass_fields__:
        cp = dataclasses.replace(cp, needs_layout_passes=False)
    if "use_tc_tiling_on_sc" in pltpu.CompilerParams.__dataclass_fields__:
        cp = dataclasses.replace(cp, use_tc_tiling_on_sc=False)
    return pl.kernel(
        _sc_body,
        out_type=jax.ShapeDtypeStruct((NROWS, HD), jnp.float32),
        mesh=mesh,
        scratch_types=[
            pltpu.VMEM((2 * HPKW,), jnp.int32),
            pltpu.VMEM((2 * HPKW,), jnp.int32),
            pltpu.VMEM((TPC, HD), jnp.float32),
            pltpu.VMEM((TPC, HD), jnp.float32),
            pltpu.VMEM((CR, HD), jnp.float32),
            pltpu.VMEM((CR, HD), jnp.float32),
            pltpu.SemaphoreType.DMA,
            pltpu.SemaphoreType.DMA,
            pltpu.SemaphoreType.DMA,
            pltpu.SemaphoreType.DMA,
            pltpu.SemaphoreType.DMA,
            pltpu.SemaphoreType.DMA,
        ],
        compiler_params=cp,
    )


def _out_body(rows, wout, bout, o_ref):
    ot = lax.dot_general(wout[...], rows[...].astype(jnp.bfloat16),
                         dimension_numbers=(((0,), (1,)), ((), ())),
                         preferred_element_type=jnp.float32)
    o_ref[0] = ot + bout[...]


def _out_call(rows, wout, bout):
    return pl.pallas_call(
        _out_body,
        grid=(BB * NBLK,),
        in_specs=[
            pl.BlockSpec((TB, CC), lambda i: (i, 0)),
            _full_spec(wout.shape), _full_spec(bout.shape),
        ],
        out_specs=pl.BlockSpec((1, CC, TB), lambda i: (i // NBLK, 0, i % NBLK)),
        out_shape=jax.ShapeDtypeStruct((BB, CC, NQ), jnp.float32),
    )(rows, wout, bout)


@jax.jit
def kernel(x, W_off, b_off, W_attn, b_attn, W_val, b_val, W_out, b_out):
    xt = x.reshape(BB, CC, NQ)
    value, packed = _prep_call(
        xt, W_val.astype(jnp.bfloat16), b_val.reshape(1, CC),
        W_off.astype(jnp.bfloat16), b_off.reshape(1, -1),
        W_attn.astype(jnp.bfloat16), b_attn.reshape(1, -1))
    sampled = _sc_gather_fn()(value.reshape(NROWS, HD), packed.reshape(-1))
    out_t = _out_call(sampled.reshape(NT, CC), W_out.astype(jnp.bfloat16),
                      b_out.reshape(CC, 1))
    return out_t.reshape(BB, CC, ZZ, HH, WW)


# speedup vs baseline: 1.3829x; 1.1526x over previous
import dataclasses
import functools

import numpy as np
import jax
import jax.numpy as jnp
from jax import lax
from jax.experimental import pallas as pl
from jax.experimental.pallas import tpu as pltpu
from jax.experimental.pallas import tpu_sc as plsc

BB, CC, ZZ, HH, WW = 2, 128, 8, 48, 48
HEADS, POINTS = 8, 4
GH, GW = ZZ * HH, WW
NQ = GH * GW
NT = BB * NQ
HD = CC // HEADS
TAPS = POINTS * 4
NROWS = NT * HEADS

TB = 1024
NBLK = NQ // TB

NQH = NQ * HEADS
NWORK = 32
RW = NROWS // NWORK
TKW = NT // NWORK
CR = 128
TPB = CR // HEADS
TPC = CR * TAPS
NGATH = TPC // 128
NCH = RW // CR
HPKW = TPB * CC


def _lane_consts():
    px = np.zeros((2 * HEADS * POINTS, HEADS * TAPS), np.float32)
    py = np.zeros_like(px)
    rexp = np.zeros((HEADS * POINTS, HEADS * TAPS), np.float32)
    for i in range(HEADS * POINTS):
        for c in range(4):
            l = i * 4 + c
            px[2 * i, l] = 1.0
            py[2 * i + 1, l] = 1.0
            rexp[i, l] = 1.0
    gsum = np.zeros((HEADS * POINTS, HEADS * POINTS), np.float32)
    for i in range(HEADS * POINTS):
        for j in range(HEADS * POINTS):
            if i // POINTS == j // POINTS:
                gsum[i, j] = 1.0
    lanes = np.arange(HEADS * TAPS)
    corner = lanes % 4
    oxl = (corner % 2).astype(np.float32)[None, :]
    oyl = (corner // 2).astype(np.float32)[None, :]
    hl = (lanes // TAPS).astype(np.float32)[None, :]
    return px, py, rexp, gsum, oxl, oyl, hl


_PX, _PY, _REXP, _GSUM, _OXL, _OYL, _HL = _lane_consts()


def _prep_body(xt, wv, bv, wo, bo, wa, ba, pxm, pym, rexp, gsum, oxl, oyl, hl,
               val_o, pk_o):
    i = pl.program_id(0)
    b = i // NBLK
    qbase = (i % NBLK) * TB
    xb = xt[0]
    dot = functools.partial(lax.dot_general,
                            precision=lax.Precision.HIGHEST,
                            preferred_element_type=jnp.float32)
    dnT = (((0,), (0,)), ((), ()))
    dnN = (((1,), (0,)), ((), ()))
    dot16 = functools.partial(lax.dot_general,
                              preferred_element_type=jnp.float32)
    xb16 = xb.astype(jnp.bfloat16)
    val_o[...] = dot16(xb16, wv[...], dimension_numbers=dnT) + bv[...]
    off = dot16(xb16, wo[...], dimension_numbers=dnT) + bo[...]
    logits = dot16(xb16, wa[...], dimension_numbers=dnT) + ba[...]
    e = jnp.exp(logits)
    s = dot16(e.astype(jnp.bfloat16), gsum[...],
              dimension_numbers=dnN)
    attn = e / s
    attn128 = dot16(attn.astype(jnp.bfloat16), rexp[...],
                    dimension_numbers=dnN)
    off16 = off.astype(jnp.bfloat16)
    offx = dot16(off16, pxm[...], dimension_numbers=dnN)
    offy = dot16(off16, pym[...], dimension_numbers=dnN)
    q = qbase + lax.broadcasted_iota(jnp.int32, (TB, 1), 0)
    iw = (q % GW).astype(jnp.float32)
    ihw = (q // GW).astype(jnp.float32)
    px = offx + iw
    py = offy + ihw
    x0 = jnp.floor(px)
    y0 = jnp.floor(py)
    dx = px - x0
    dy = py - y0
    ox = oxl[...]
    oy = oyl[...]
    xi = x0 + ox
    yi = y0 + oy
    valid = ((xi >= 0) & (xi <= GW - 1) & (yi >= 0) & (yi <= GH - 1))
    xic = jnp.clip(xi, 0, GW - 1)
    yic = jnp.clip(yi, 0, GH - 1)
    lin = yic * GW + xic
    gidx = lin * HEADS + hl[...] + (b * NQH).astype(jnp.float32)
    pk_o[:TB] = gidx.astype(jnp.int32)
    wx = ox * dx + (1.0 - ox) * (1.0 - dx)
    wy = oy * dy + (1.0 - oy) * (1.0 - dy)
    wt = wx * wy * valid.astype(jnp.float32) * attn128
    pk_o[TB:] = lax.bitcast_convert_type(wt, jnp.int32)


def _full_spec(shape):
    nd = len(shape)
    return pl.BlockSpec(shape, lambda i: (0,) * nd)


def _prep_call(xt, wv, bv, wo, bo, wa, ba):
    consts = [jnp.asarray(a, dtype=jnp.bfloat16) for a in
              (_PX, _PY, _REXP, _GSUM)]
    consts += [jnp.asarray(a) for a in (_OXL, _OYL, _HL)]
    return pl.pallas_call(
        _prep_body,
        grid=(BB * NBLK,),
        in_specs=[
            pl.BlockSpec((1, CC, TB), lambda i: (i // NBLK, 0, i % NBLK)),
            _full_spec(wv.shape), _full_spec(bv.shape),
            _full_spec(wo.shape), _full_spec(bo.shape),
            _full_spec(wa.shape), _full_spec(ba.shape),
        ] + [_full_spec(c.shape) for c in consts],
        out_specs=[
            pl.BlockSpec((TB, CC), lambda i: (i, 0)),
            pl.BlockSpec((2 * TB, CC), lambda i: (i, 0)),
        ],
        out_shape=[
            jax.ShapeDtypeStruct((NT, CC), jnp.float32),
            jax.ShapeDtypeStruct((2 * NT, CC), jnp.int32),
        ],
    )(xt, wv, bv, wo, bo, wa, ba, *consts)


def _bcast(vec, j):
    return vec.at[jnp.full((HD,), j, jnp.int32)].get(mode="promise_in_bounds")


def _sc_body(val_hbm, pk_hbm, out_hbm,
             ld0, ld1, buf0, buf1, out0, out1,
             ls0, ls1, gs0, gs1, os0, os1):
    wid = lax.axis_index("s") * 2 + lax.axis_index("c")
    row_base = wid * RW
    tok_base = wid * TKW
    LD, BUF, OUT = (ld0, ld1), (buf0, buf1), (out0, out1)
    LS, GS, OS = (ls0, ls1), (gs0, gs1), (os0, os1)

    def issue_load(i, p):
        t0 = tok_base + i * TPB
        g = t0 // TB
        r = t0 % TB
        idx_off = (g * 2 * TB + r) * CC
        wt_off = (g * 2 * TB + TB + r) * CC
        pltpu.async_copy(pk_hbm.at[pl.ds(idx_off, HPKW)],
                         LD[p].at[pl.ds(0, HPKW)], LS[p])
        pltpu.async_copy(pk_hbm.at[pl.ds(wt_off, HPKW)],
                         LD[p].at[pl.ds(HPKW, HPKW)], LS[p])

    def wait_load(p):
        pltpu.make_async_copy(pk_hbm.at[pl.ds(0, 2 * HPKW)], LD[p],
                              LS[p]).wait()

    def issue_gathers(p):
        for g in range(NGATH):
            pltpu.async_copy(val_hbm.at[LD[p].at[pl.ds(g * 128, 128)]],
                             BUF[p].at[pl.ds(g * 128, 128)], GS[p])

    def wait_gathers(p):
        pltpu.make_async_copy(val_hbm.at[pl.ds(0, TPC)], BUF[p], GS[p]).wait()

    def wait_store(p):
        pltpu.make_async_copy(OUT[p], out_hbm.at[pl.ds(0, CR)], OS[p]).wait()

    def compute_store(i, p):
        @pl.loop(0, TPB)
        def _tok(m):
            ldb = HPKW + m * CC
            bufb = m * CC
            for h in range(HEADS):
                w16 = plsc.bitcast(LD[p][pl.ds(ldb + h * HD, HD)],
                                   jnp.float32)
                a0 = _bcast(w16, 0) * BUF[p][bufb + h * HD + 0]
                a1 = _bcast(w16, 1) * BUF[p][bufb + h * HD + 1]
                a2 = _bcast(w16, 2) * BUF[p][bufb + h * HD + 2]
                a3 = _bcast(w16, 3) * BUF[p][bufb + h * HD + 3]
                for j in range(4, TAPS, 4):
                    a0 = a0 + _bcast(w16, j) * BUF[p][bufb + h * HD + j]
                    a1 = a1 + _bcast(w16, j + 1) * BUF[p][bufb + h * HD + j + 1]
                    a2 = a2 + _bcast(w16, j + 2) * BUF[p][bufb + h * HD + j + 2]
                    a3 = a3 + _bcast(w16, j + 3) * BUF[p][bufb + h * HD + j + 3]
                OUT[p][m * HEADS + h] = (a0 + a1) + (a2 + a3)

        pltpu.async_copy(OUT[p], out_hbm.at[pl.ds(row_base + i * CR, CR)],
                         OS[p])

    issue_load(0, 0)
    issue_load(1, 1)
    wait_load(0)
    issue_gathers(0)

    @pl.loop(0, NCH // 2)
    def _pair(k):
        for p in (0, 1):
            i = k * 2 + p
            q = 1 - p

            @pl.when(i + 1 < NCH)
            def _():
                wait_load(q)
                issue_gathers(q)

            wait_gathers(p)

            @pl.when(i >= 2)
            def _():
                wait_store(p)

            compute_store(i, p)

            @pl.when(i + 2 < NCH)
            def _():
                issue_load(i + 2, p)

    wait_store(0)
    wait_store(1)


@functools.cache
def _sc_gather_fn():
    mesh = plsc.VectorSubcoreMesh(core_axis_name="c", subcore_axis_name="s",
                                  num_cores=2, num_subcores=16)
    cp = pltpu.CompilerParams()
    if "needs_layout_passes" in pltpu.CompilerParams.__dataclass_fields__:
        cp = dataclasses.replace(cp, needs_layout_passes=False)
    if "use_tc_tiling_on_sc" in pltpu.CompilerParams.__dataclass_fields__:
        cp = dataclasses.replace(cp, use_tc_tiling_on_sc=False)
    return pl.kernel(
        _sc_body,
        out_type=jax.ShapeDtypeStruct((NROWS, HD), jnp.float32),
        mesh=mesh,
        scratch_types=[
            pltpu.VMEM((2 * HPKW,), jnp.int32),
            pltpu.VMEM((2 * HPKW,), jnp.int32),
            pltpu.VMEM((TPC, HD), jnp.float32),
            pltpu.VMEM((TPC, HD), jnp.float32),
            pltpu.VMEM((CR, HD), jnp.float32),
            pltpu.VMEM((CR, HD), jnp.float32),
            pltpu.SemaphoreType.DMA,
            pltpu.SemaphoreType.DMA,
            pltpu.SemaphoreType.DMA,
            pltpu.SemaphoreType.DMA,
            pltpu.SemaphoreType.DMA,
            pltpu.SemaphoreType.DMA,
        ],
        compiler_params=cp,
    )


def _out_body(rows, wout, bout, o_ref):
    ot = lax.dot_general(wout[...], rows[...].astype(jnp.bfloat16),
                         dimension_numbers=(((0,), (1,)), ((), ())),
                         preferred_element_type=jnp.float32)
    o_ref[0] = ot + bout[...]


def _out_call(rows, wout, bout):
    return pl.pallas_call(
        _out_body,
        grid=(BB * NBLK,),
        in_specs=[
            pl.BlockSpec((TB, CC), lambda i: (i, 0)),
            _full_spec(wout.shape), _full_spec(bout.shape),
        ],
        out_specs=pl.BlockSpec((1, CC, TB), lambda i: (i // NBLK, 0, i % NBLK)),
        out_shape=jax.ShapeDtypeStruct((BB, CC, NQ), jnp.float32),
    )(rows, wout, bout)


@jax.jit
def kernel(x, W_off, b_off, W_attn, b_attn, W_val, b_val, W_out, b_out):
    xt = x.reshape(BB, CC, NQ)
    value, packed = _prep_call(
        xt, W_val.astype(jnp.bfloat16), b_val.reshape(1, CC),
        W_off.astype(jnp.bfloat16), b_off.reshape(1, -1),
        W_attn.astype(jnp.bfloat16), b_attn.reshape(1, -1))
    sampled = _sc_gather_fn()(value.reshape(NROWS, HD), packed.reshape(-1))
    out_t = _out_call(sampled.reshape(NT, CC), W_out.astype(jnp.bfloat16),
                      b_out.reshape(CC, 1))
    return out_t.reshape(BB, CC, ZZ, HH, WW)


# speedup vs baseline: 1.3835x; 1.0004x over previous
import dataclasses
import functools

import numpy as np
import jax
import jax.numpy as jnp
from jax import lax
from jax.experimental import pallas as pl
from jax.experimental.pallas import tpu as pltpu
from jax.experimental.pallas import tpu_sc as plsc

BB, CC, ZZ, HH, WW = 2, 128, 8, 48, 48
HEADS, POINTS = 8, 4
GH, GW = ZZ * HH, WW
NQ = GH * GW
NT = BB * NQ
HD = CC // HEADS
TAPS = POINTS * 4
NROWS = NT * HEADS

TB = 1024
NBLK = NQ // TB

NQH = NQ * HEADS
NWORK = 32
RW = NROWS // NWORK
TKW = NT // NWORK
CR = 128
TPB = CR // HEADS
TPC = CR * TAPS
NGATH = TPC // 128
NCH = RW // CR
HPKW = TPB * CC


def _lane_consts():
    px = np.zeros((2 * HEADS * POINTS, HEADS * TAPS), np.float32)
    py = np.zeros_like(px)
    rexp = np.zeros((HEADS * POINTS, HEADS * TAPS), np.float32)
    for i in range(HEADS * POINTS):
        for c in range(4):
            l = i * 4 + c
            px[2 * i, l] = 1.0
            py[2 * i + 1, l] = 1.0
            rexp[i, l] = 1.0
    gsum = np.zeros((HEADS * POINTS, HEADS * POINTS), np.float32)
    for i in range(HEADS * POINTS):
        for j in range(HEADS * POINTS):
            if i // POINTS == j // POINTS:
                gsum[i, j] = 1.0
    lanes = np.arange(HEADS * TAPS)
    corner = lanes % 4
    oxl = (corner % 2).astype(np.float32)[None, :]
    oyl = (corner // 2).astype(np.float32)[None, :]
    hl = (lanes // TAPS).astype(np.float32)[None, :]
    return px, py, rexp, gsum, oxl, oyl, hl


_PX, _PY, _REXP, _GSUM, _OXL, _OYL, _HL = _lane_consts()


def _prep_body(xt, wv, bv, wo, bo, wa, ba, pxm, pym, rexp, gsum, oxl, oyl, hl,
               val_o, pk_o):
    i = pl.program_id(0)
    b = i // NBLK
    qbase = (i % NBLK) * TB
    xb = xt[0]
    dnT = (((0,), (0,)), ((), ()))
    dnN = (((1,), (0,)), ((), ()))
    dot16 = functools.partial(lax.dot_general,
                              preferred_element_type=jnp.float32)
    xb16 = xb.astype(jnp.bfloat16)
    val_o[...] = dot16(xb16, wv[...], dimension_numbers=dnT) + bv[...]
    off = dot16(xb16, wo[...], dimension_numbers=dnT) + bo[...]
    logits = dot16(xb16, wa[...], dimension_numbers=dnT) + ba[...]
    e = jnp.exp(logits)
    s = dot16(e.astype(jnp.bfloat16), gsum[...],
              dimension_numbers=dnN)
    attn = e / s
    attn128 = dot16(attn.astype(jnp.bfloat16), rexp[...],
                    dimension_numbers=dnN)
    off16 = off.astype(jnp.bfloat16)
    offx = dot16(off16, pxm[...], dimension_numbers=dnN)
    offy = dot16(off16, pym[...], dimension_numbers=dnN)
    q = qbase + lax.broadcasted_iota(jnp.int32, (TB, 1), 0)
    iw = (q % GW).astype(jnp.float32)
    ihw = (q // GW).astype(jnp.float32)
    px = offx + iw
    py = offy + ihw
    x0 = jnp.floor(px)
    y0 = jnp.floor(py)
    dx = px - x0
    dy = py - y0
    ox = oxl[...]
    oy = oyl[...]
    xi = x0 + ox
    yi = y0 + oy
    valid = ((xi >= 0) & (xi <= GW - 1) & (yi >= 0) & (yi <= GH - 1))
    xic = jnp.clip(xi, 0, GW - 1)
    yic = jnp.clip(yi, 0, GH - 1)
    lin = yic * GW + xic
    gidx = lin * HEADS + hl[...] + (b * NQH).astype(jnp.float32)
    pk_o[:TB] = gidx.astype(jnp.int32)
    wx = ox * dx + (1.0 - ox) * (1.0 - dx)
    wy = oy * dy + (1.0 - oy) * (1.0 - dy)
    wt = wx * wy * valid.astype(jnp.float32) * attn128
    pk_o[TB:] = lax.bitcast_convert_type(wt, jnp.int32)


def _full_spec(shape):
    nd = len(shape)
    return pl.BlockSpec(shape, lambda i: (0,) * nd)


def _prep_call(xt, wv, bv, wo, bo, wa, ba):
    consts = [jnp.asarray(a, dtype=jnp.bfloat16) for a in
              (_PX, _PY, _REXP, _GSUM)]
    consts += [jnp.asarray(a) for a in (_OXL, _OYL, _HL)]
    return pl.pallas_call(
        _prep_body,
        grid=(BB * NBLK,),
        in_specs=[
            pl.BlockSpec((1, CC, TB), lambda i: (i // NBLK, 0, i % NBLK)),
            _full_spec(wv.shape), _full_spec(bv.shape),
            _full_spec(wo.shape), _full_spec(bo.shape),
            _full_spec(wa.shape), _full_spec(ba.shape),
        ] + [_full_spec(c.shape) for c in consts],
        out_specs=[
            pl.BlockSpec((TB, CC), lambda i: (i, 0)),
            pl.BlockSpec((2 * TB, CC), lambda i: (i, 0)),
        ],
        out_shape=[
            jax.ShapeDtypeStruct((NT, CC), jnp.float32),
            jax.ShapeDtypeStruct((2 * NT, CC), jnp.int32),
        ],
    )(xt, wv, bv, wo, bo, wa, ba, *consts)


def _bcast(vec, j):
    return vec.at[jnp.full((HD,), j, jnp.int32)].get(mode="promise_in_bounds")


def _sc_body(val_hbm, pk_hbm, out_hbm,
             ld0, ld1, buf0, buf1, out0, out1,
             ls0, ls1, gs0, gs1, os0, os1):
    wid = lax.axis_index("s") * 2 + lax.axis_index("c")
    row_base = wid * RW
    tok_base = wid * TKW
    LD, BUF, OUT = (ld0, ld1), (buf0, buf1), (out0, out1)
    LS, GS, OS = (ls0, ls1), (gs0, gs1), (os0, os1)

    def issue_load(i, p):
        t0 = tok_base + i * TPB
        g = t0 // TB
        r = t0 % TB
        idx_off = (g * 2 * TB + r) * CC
        wt_off = (g * 2 * TB + TB + r) * CC
        pltpu.async_copy(pk_hbm.at[pl.ds(idx_off, HPKW)],
                         LD[p].at[pl.ds(0, HPKW)], LS[p])
        pltpu.async_copy(pk_hbm.at[pl.ds(wt_off, HPKW)],
                         LD[p].at[pl.ds(HPKW, HPKW)], LS[p])

    def wait_load(p):
        pltpu.make_async_copy(pk_hbm.at[pl.ds(0, 2 * HPKW)], LD[p],
                              LS[p]).wait()

    def issue_gathers(p):
        for g in range(NGATH):
            pltpu.async_copy(val_hbm.at[LD[p].at[pl.ds(g * 128, 128)]],
                             BUF[p].at[pl.ds(g * 128, 128)], GS[p])

    def wait_gathers(p):
        pltpu.make_async_copy(val_hbm.at[pl.ds(0, TPC)], BUF[p], GS[p]).wait()

    def wait_store(p):
        pltpu.make_async_copy(OUT[p], out_hbm.at[pl.ds(0, CR)], OS[p]).wait()

    def compute_store(i, p):
        @pl.loop(0, TPB)
        def _tok(m):
            ldb = HPKW + m * CC
            bufb = m * CC
            for h in range(HEADS):
                w16 = plsc.bitcast(LD[p][pl.ds(ldb + h * HD, HD)],
                                   jnp.float32)
                a0 = _bcast(w16, 0) * BUF[p][bufb + h * HD + 0]
                a1 = _bcast(w16, 1) * BUF[p][bufb + h * HD + 1]
                a2 = _bcast(w16, 2) * BUF[p][bufb + h * HD + 2]
                a3 = _bcast(w16, 3) * BUF[p][bufb + h * HD + 3]
                for j in range(4, TAPS, 4):
                    a0 = a0 + _bcast(w16, j) * BUF[p][bufb + h * HD + j]
                    a1 = a1 + _bcast(w16, j + 1) * BUF[p][bufb + h * HD + j + 1]
                    a2 = a2 + _bcast(w16, j + 2) * BUF[p][bufb + h * HD + j + 2]
                    a3 = a3 + _bcast(w16, j + 3) * BUF[p][bufb + h * HD + j + 3]
                OUT[p][m * HEADS + h] = (a0 + a1) + (a2 + a3)

        pltpu.async_copy(OUT[p], out_hbm.at[pl.ds(row_base + i * CR, CR)],
                         OS[p])

    issue_load(0, 0)
    issue_load(1, 1)
    wait_load(0)
    issue_gathers(0)

    @pl.loop(0, NCH // 2)
    def _pair(k):
        for p in (0, 1):
            i = k * 2 + p
            q = 1 - p

            @pl.when(i + 1 < NCH)
            def _():
                wait_load(q)
                issue_gathers(q)

            wait_gathers(p)

            @pl.when(i >= 2)
            def _():
                wait_store(p)

            compute_store(i, p)

            @pl.when(i + 2 < NCH)
            def _():
                issue_load(i + 2, p)

    wait_store(0)
    wait_store(1)


@functools.cache
def _sc_gather_fn():
    mesh = plsc.VectorSubcoreMesh(core_axis_name="c", subcore_axis_name="s",
                                  num_cores=2, num_subcores=16)
    cp = pltpu.CompilerParams()
    if "needs_layout_passes" in pltpu.CompilerParams.__dataclass_fields__:
        cp = dataclasses.replace(cp, needs_layout_passes=False)
    if "use_tc_tiling_on_sc" in pltpu.CompilerParams.__dataclass_fields__:
        cp = dataclasses.replace(cp, use_tc_tiling_on_sc=False)
    return pl.kernel(
        _sc_body,
        out_type=jax.ShapeDtypeStruct((NROWS, HD), jnp.float32),
        mesh=mesh,
        scratch_types=[
            pltpu.VMEM((2 * HPKW,), jnp.int32),
            pltpu.VMEM((2 * HPKW,), jnp.int32),
            pltpu.VMEM((TPC, HD), jnp.float32),
            pltpu.VMEM((TPC, HD), jnp.float32),
            pltpu.VMEM((CR, HD), jnp.float32),
            pltpu.VMEM((CR, HD), jnp.float32),
            pltpu.SemaphoreType.DMA,
            pltpu.SemaphoreType.DMA,
            pltpu.SemaphoreType.DMA,
            pltpu.SemaphoreType.DMA,
            pltpu.SemaphoreType.DMA,
            pltpu.SemaphoreType.DMA,
        ],
        compiler_params=cp,
    )


def _out_body(rows, wout, bout, o_ref):
    ot = lax.dot_general(wout[...], rows[...].astype(jnp.bfloat16),
                         dimension_numbers=(((0,), (1,)), ((), ())),
                         preferred_element_type=jnp.float32)
    o_ref[0] = ot + bout[...]


def _out_call(rows, wout, bout):
    return pl.pallas_call(
        _out_body,
        grid=(BB * NBLK,),
        in_specs=[
            pl.BlockSpec((TB, CC), lambda i: (i, 0)),
            _full_spec(wout.shape), _full_spec(bout.shape),
        ],
        out_specs=pl.BlockSpec((1, CC, TB), lambda i: (i // NBLK, 0, i % NBLK)),
        out_shape=jax.ShapeDtypeStruct((BB, CC, NQ), jnp.float32),
    )(rows, wout, bout)


@jax.jit
def kernel(x, W_off, b_off, W_attn, b_attn, W_val, b_val, W_out, b_out):
    xt = x.reshape(BB, CC, NQ)
    value, packed = _prep_call(
        xt, W_val.astype(jnp.bfloat16), b_val.reshape(1, CC),
        W_off.astype(jnp.bfloat16), b_off.reshape(1, -1),
        W_attn.astype(jnp.bfloat16), b_attn.reshape(1, -1))
    sampled = _sc_gather_fn()(value.reshape(NROWS, HD), packed.reshape(-1))
    out_t = _out_call(sampled.reshape(NT, CC), W_out.astype(jnp.bfloat16),
                      b_out.reshape(CC, 1))
    return out_t.reshape(BB, CC, ZZ, HH, WW)
